# baseline (speedup 1.0000x reference)
import jax
from jax.experimental import pallas as pl
from jax.experimental.pallas import tpu as pltpu


_N_CHUNKS = 8
_ROWS = 320000
_CHUNK_ROWS = _ROWS // _N_CHUNKS


def _copy_kernel(x_ref, o_ref, sems):
    for i in range(_N_CHUNKS):
        sl = pl.ds(i * _CHUNK_ROWS, _CHUNK_ROWS)
        pltpu.make_async_copy(x_ref.at[sl], o_ref.at[sl], sems.at[i]).start()
    for i in range(_N_CHUNKS):
        sl = pl.ds(i * _CHUNK_ROWS, _CHUNK_ROWS)
        pltpu.make_async_copy(x_ref.at[sl], o_ref.at[sl], sems.at[i]).wait()


def kernel(x_j):
    return pl.pallas_call(
        _copy_kernel,
        out_shape=jax.ShapeDtypeStruct(x_j.shape, x_j.dtype),
        in_specs=[pl.BlockSpec(memory_space=pl.ANY)],
        out_specs=pl.BlockSpec(memory_space=pl.ANY),
        scratch_shapes=[pltpu.SemaphoreType.DMA((_N_CHUNKS,))],
    )(x_j)


# speedup vs baseline: 48.2680x; 48.2680x over previous
import jax
from jax.experimental import pallas as pl
from jax.experimental.pallas import tpu as pltpu


_ROWS = 320000
_BLOCK_ROWS = 8000


def _copy_kernel(x_ref, o_ref):
    o_ref[...] = x_ref[...]


def kernel(x_j):
    grid = (_ROWS // _BLOCK_ROWS,)
    return pl.pallas_call(
        _copy_kernel,
        out_shape=jax.ShapeDtypeStruct(x_j.shape, x_j.dtype),
        grid=grid,
        in_specs=[pl.BlockSpec((_BLOCK_ROWS, 128), lambda i: (i, 0))],
        out_specs=pl.BlockSpec((_BLOCK_ROWS, 128), lambda i: (i, 0)),
    )(x_j)
